# baseline (speedup 1.0000x reference)
import jax
import jax.numpy as jnp
from jax import lax
from jax.experimental import pallas as pl
from jax.experimental.pallas import tpu as pltpu

_BN_EPS = 1e-5
_NORM_EPS = 1e-12
_DIFFPOOL_EPS = 1e-15
_VMEM_LIMIT = 48 * 1024 * 1024


def _inv_deg(adj):
    return 1.0 / jnp.maximum(jnp.sum(adj, axis=-1, keepdims=True), 1.0)


def _l2norm(out):
    ss = jnp.sum(out * out, axis=-1, keepdims=True)
    return out * lax.rsqrt(jnp.maximum(ss, _NORM_EPS * _NORM_EPS))


def _sage(cat, w_ref, b):
    out = jnp.dot(cat, w_ref[...], preferred_element_type=jnp.float32) + b
    return _l2norm(out)


def _bn_coef(st_ref, idx, inv_bn, w, b):
    s = jnp.sum(st_ref[...], axis=0)
    mean = s[2 * idx:2 * idx + 1] * inv_bn
    var = jnp.maximum(s[2 * idx + 1:2 * idx + 2] * inv_bn - mean * mean, 0.0)
    return mean, lax.rsqrt(var + _BN_EPS) * w, b


def _stage1_body(x_ref, adj_ref, w1p_ref, w1e_ref, vec_ref,
                 r1p_ref, r1e_ref, adj8_ref, st_ref):
    adj = adj_ref[...]
    x = x_ref[...]
    adj8_ref[...] = adj.astype(jnp.int8)
    sum_adj2 = jnp.sum(jnp.sum(adj * adj, axis=1, keepdims=True),
                       axis=0, keepdims=True)
    agg = jnp.dot(adj, x, preferred_element_type=jnp.float32) * _inv_deg(adj)
    cat = jnp.concatenate([agg, x], axis=-1)
    vec = vec_ref[...]
    r1p = jnp.maximum(_sage(cat, w1p_ref, vec[0:1]), 0.0)
    r1e = jnp.maximum(_sage(cat, w1e_ref, vec[1:2]), 0.0)
    r1p_ref[...] = r1p
    r1e_ref[...] = r1e.astype(jnp.bfloat16)
    H = r1p.shape[1]
    st_ref[...] = jnp.concatenate([
        jnp.sum(r1p, axis=0, keepdims=True),
        jnp.sum(r1p * r1p, axis=0, keepdims=True),
        jnp.sum(r1e, axis=0, keepdims=True),
        jnp.sum(r1e * r1e, axis=0, keepdims=True),
        sum_adj2 * jnp.ones((1, H), jnp.float32),
        jnp.zeros((3, H), jnp.float32)], axis=0)


def _stage2_body(adj8_ref, r1p_ref, r1e_ref, st1_ref, w2p_ref, w2e_ref,
                 vec_ref, r2p_ref, r2e_ref, st_ref):
    vec = vec_ref[...]
    inv_bn = 1.0 / float(st1_ref.shape[0] * r1p_ref.shape[0])
    mp, rwp, bbp = _bn_coef(st1_ref, 0, inv_bn, vec[0:1], vec[1:2])
    me, rwe, bbe = _bn_coef(st1_ref, 1, inv_bn, vec[2:3], vec[3:4])
    h1p = (r1p_ref[...] - mp) * rwp + bbp
    h1e = (r1e_ref[...].astype(jnp.float32) - me) * rwe + bbe
    adj = adj8_ref[...].astype(jnp.float32)
    agg = jnp.dot(adj, jnp.concatenate([h1p, h1e], axis=-1),
                  preferred_element_type=jnp.float32) * _inv_deg(adj)
    H = h1p.shape[1]
    r2p = jnp.maximum(_sage(jnp.concatenate([agg[:, :H], h1p], axis=-1),
                            w2p_ref, vec[4:5]), 0.0)
    r2e = jnp.maximum(_sage(jnp.concatenate([agg[:, H:], h1e], axis=-1),
                            w2e_ref, vec[5:6]), 0.0)
    r2p_ref[...] = r2p
    r2e_ref[...] = r2e.astype(jnp.bfloat16)
    st_ref[...] = jnp.concatenate([
        jnp.sum(r2p, axis=0, keepdims=True),
        jnp.sum(r2p * r2p, axis=0, keepdims=True),
        jnp.sum(r2e, axis=0, keepdims=True),
        jnp.sum(r2e * r2e, axis=0, keepdims=True),
        jnp.zeros((4, H), jnp.float32)], axis=0)


def _stage3_body(adj8_ref, r1p_ref, r1e_ref, r2p_ref, r2e_ref,
                 st1_ref, st2_ref, w3p_ref, w3e_ref, wlin_ref, vec_ref,
                 gd_ref,
                 out_x_ref, out_adj_ref, s_ref, link_ref, ent_ref):
    vec = vec_ref[...]
    inv_bn = 1.0 / float(st1_ref.shape[0] * r1p_ref.shape[0])
    mp, rwp, bbp = _bn_coef(st1_ref, 0, inv_bn, vec[0:1], vec[1:2])
    me, rwe, bbe = _bn_coef(st1_ref, 1, inv_bn, vec[2:3], vec[3:4])
    h1p = (r1p_ref[...] - mp) * rwp + bbp
    h1e = (r1e_ref[...].astype(jnp.float32) - me) * rwe + bbe
    mp, rwp, bbp = _bn_coef(st2_ref, 0, inv_bn, vec[4:5], vec[5:6])
    me, rwe, bbe = _bn_coef(st2_ref, 1, inv_bn, vec[6:7], vec[7:8])
    h2p = (r2p_ref[...] - mp) * rwp + bbp
    h2e = (r2e_ref[...].astype(jnp.float32) - me) * rwe + bbe

    adj = adj8_ref[...].astype(jnp.float32)
    agg = jnp.dot(adj, jnp.concatenate([h2p, h2e], axis=-1),
                  preferred_element_type=jnp.float32) * _inv_deg(adj)
    H = h2p.shape[1]
    h3p = _sage(jnp.concatenate([agg[:, :H], h2p], axis=-1), w3p_ref,
                vec[8:9])
    h3e = _sage(jnp.concatenate([agg[:, H:], h2e], axis=-1), w3e_ref,
                vec[9:10])

    logits = (jnp.dot(jnp.concatenate([h1p, h2p, h3p], axis=-1),
                      wlin_ref[...], preferred_element_type=jnp.float32)
              + vec[10:11])
    m = jnp.max(logits, axis=-1, keepdims=True)
    e = jnp.exp(logits - m)
    sb = e / jnp.sum(e, axis=-1, keepdims=True)
    s_ref[...] = sb

    xb = jnp.concatenate([h1e, h2e, h3e], axis=-1)
    cT = (((0,), (0,)), ((), ()))
    out_x_ref[...] = lax.dot_general(sb, xb, cT,
                                     preferred_element_type=jnp.float32)
    sta = lax.dot_general(sb, adj, cT, preferred_element_type=jnp.float32)
    pooled = jnp.dot(sta, sb, preferred_element_type=jnp.float32)
    sts = lax.dot_general(sb, sb, cT, preferred_element_type=jnp.float32)

    K = sb.shape[1]
    row = lax.broadcasted_iota(jnp.int32, (K, K), 0)
    col = lax.broadcasted_iota(jnp.int32, (K, K), 1)
    diag = row == col

    tr_pooled = jnp.sum(jnp.sum(jnp.where(diag, pooled, 0.0),
                                axis=1, keepdims=True),
                        axis=0, keepdims=True)
    sum_sts2 = jnp.sum(jnp.sum(sts * sts, axis=1, keepdims=True),
                       axis=0, keepdims=True)
    link_ref[...] = sum_sts2 - 2.0 * tr_pooled

    ent = -sb * jnp.log(sb + _DIFFPOOL_EPS)
    ent_ref[...] = jnp.sum(jnp.sum(ent, axis=1, keepdims=True),
                           axis=0, keepdims=True)

    mn = jnp.min(jnp.min(pooled, axis=1, keepdims=True), axis=0, keepdims=True)
    mx = jnp.max(jnp.max(pooled, axis=1, keepdims=True), axis=0, keepdims=True)
    an = (pooled - mn) / jnp.maximum(mx - mn, 1e-12)
    hard = jnp.where(an + gd_ref[...] >= 1.0 - an, 1.0, 0.0)
    ut = jnp.where(col >= row, hard, 0.0)
    sym = ut + ut.T
    out_adj_ref[...] = jnp.where(diag, 1.0, sym)


def _full(shape):
    return pl.BlockSpec(shape, lambda b: (0,) * len(shape))


def _bat(*shape):
    return pl.BlockSpec((None,) + shape, lambda b: (b,) + (0,) * len(shape))


def _params():
    return pltpu.CompilerParams(
        dimension_semantics=("parallel",),
        vmem_limit_bytes=_VMEM_LIMIT)


def kernel(x, adj, rng, pool_w_rel1, pool_b1, pool_w_root1, pool_w_rel2,
           pool_b2, pool_w_root2, pool_w_rel3, pool_b3, pool_w_root3,
           pool_bn1_w, pool_bn1_b, pool_bn2_w, pool_bn2_b, pool_w_lin,
           pool_b_lin, emb_w_rel1, emb_b1, emb_w_root1, emb_w_rel2, emb_b2,
           emb_w_root2, emb_w_rel3, emb_b3, emb_w_root3, emb_bn1_w,
           emb_bn1_b, emb_bn2_w, emb_bn2_b):
    B, N, C = x.shape
    H = pool_w_rel1.shape[1]
    K = pool_w_lin.shape[1]
    Fe = emb_w_rel3.shape[1]
    D = 2 * H + Fe

    key = jax.random.wrap_key_data(rng)
    g = jax.random.gumbel(key, (2, B, K, K), jnp.float32)
    gd = g[0] - g[1]

    def wcat(wr, wo):
        return jnp.concatenate([wr, wo], axis=0)

    w1p = wcat(pool_w_rel1, pool_w_root1)
    w2p = wcat(pool_w_rel2, pool_w_root2)
    w3p = wcat(pool_w_rel3, pool_w_root3)
    w1e = wcat(emb_w_rel1, emb_w_root1)
    w2e = wcat(emb_w_rel2, emb_w_root2)
    w3e = wcat(emb_w_rel3, emb_w_root3)

    zrow = jnp.zeros((1, H), jnp.float32)
    vec1 = jnp.concatenate([pool_b1, emb_b1] + [zrow] * 6, axis=0)
    vec2 = jnp.concatenate([pool_bn1_w, pool_bn1_b, emb_bn1_w, emb_bn1_b,
                            pool_b2, emb_b2] + [zrow] * 2, axis=0)
    vec3 = jnp.concatenate([pool_bn1_w, pool_bn1_b, emb_bn1_w, emb_bn1_b,
                            pool_bn2_w, pool_bn2_b, emb_bn2_w, emb_bn2_b,
                            pool_b3, emb_b3, pool_b_lin] + [zrow] * 5, axis=0)

    r1p, r1e, adj8, st1 = pl.pallas_call(
        _stage1_body,
        grid=(B,),
        in_specs=[_bat(N, C), _bat(N, N),
                  _full((2 * C, H)), _full((2 * C, H)), _full((8, H))],
        out_specs=(_bat(N, H), _bat(N, H), _bat(N, N), _bat(8, H)),
        out_shape=(jax.ShapeDtypeStruct((B, N, H), jnp.float32),
                   jax.ShapeDtypeStruct((B, N, H), jnp.bfloat16),
                   jax.ShapeDtypeStruct((B, N, N), jnp.int8),
                   jax.ShapeDtypeStruct((B, 8, H), jnp.float32)),
        compiler_params=_params(),
    )(x, adj, w1p, w1e, vec1)

    r2p, r2e, st2 = pl.pallas_call(
        _stage2_body,
        grid=(B,),
        in_specs=[_bat(N, N), _bat(N, H), _bat(N, H), _full((B, 8, H)),
                  _full((2 * H, H)), _full((2 * H, H)), _full((8, H))],
        out_specs=(_bat(N, H), _bat(N, H), _bat(8, H)),
        out_shape=(jax.ShapeDtypeStruct((B, N, H), jnp.float32),
                   jax.ShapeDtypeStruct((B, N, H), jnp.bfloat16),
                   jax.ShapeDtypeStruct((B, 8, H), jnp.float32)),
        compiler_params=_params(),
    )(adj8, r1p, r1e, st1, w2p, w2e, vec2)

    out_x, new_adj, s_soft, link_p, ent_p = pl.pallas_call(
        _stage3_body,
        grid=(B,),
        in_specs=[_bat(N, N), _bat(N, H), _bat(N, H), _bat(N, H),
                  _bat(N, H), _full((B, 8, H)), _full((B, 8, H)),
                  _full((2 * H, H)), _full((2 * H, H)),
                  _full((2 * H + K, K)), _full((16, H)), _bat(K, K)],
        out_specs=(_bat(K, D), _bat(K, K), _bat(N, K), _bat(1, 1),
                   _bat(1, 1)),
        out_shape=(jax.ShapeDtypeStruct((B, K, D), jnp.float32),
                   jax.ShapeDtypeStruct((B, K, K), jnp.float32),
                   jax.ShapeDtypeStruct((B, N, K), jnp.float32),
                   jax.ShapeDtypeStruct((B, 1, 1), jnp.float32),
                   jax.ShapeDtypeStruct((B, 1, 1), jnp.float32)),
        compiler_params=_params(),
    )(adj8, r1p, r1e, r2p, r2e, st1, st2, w3p, w3e, pool_w_lin, vec3, gd)

    sum_adj2 = st1[:, 4:5, 0:1]
    link = (jnp.sqrt(jnp.maximum(jnp.sum(sum_adj2 + link_p), 0.0))
            / float(B * N * N))
    ent = jnp.sum(ent_p) / float(B * N)
    return out_x, new_adj, link, ent, s_soft


# speedup vs baseline: 1.1678x; 1.1678x over previous
import jax
import jax.numpy as jnp
from jax import lax
from jax.experimental import pallas as pl
from jax.experimental.pallas import tpu as pltpu

_BN_EPS = 1e-5
_NORM_EPS = 1e-12
_DIFFPOOL_EPS = 1e-15
_VMEM_LIMIT = 48 * 1024 * 1024


def _inv_deg(adj):
    return 1.0 / jnp.maximum(jnp.sum(adj, axis=-1, keepdims=True), 1.0)


def _l2norm(out):
    ss = jnp.sum(out * out, axis=-1, keepdims=True)
    return out * lax.rsqrt(jnp.maximum(ss, _NORM_EPS * _NORM_EPS))


def _sage(cat, w_ref, b):
    out = jnp.dot(cat, w_ref[...], preferred_element_type=jnp.float32) + b
    return _l2norm(out)


def _bn_coef(st, inv_bn, w, b):
    mean = st[0:1] * inv_bn
    var = jnp.maximum(st[1:2] * inv_bn - mean * mean, 0.0)
    return mean, lax.rsqrt(var + _BN_EPS) * w, b


def _mono_body(x_ref, adj_ref, gd_ref, w1p_ref, w1e_ref, w2p_ref, w2e_ref,
               w3p_ref, w3e_ref, wlin_ref, vec_ref,
               out_x_ref, out_adj_ref, s_ref, link_ref, ent_ref,
               adjs, r1ps, r1es, r2ps, r2es, sts, a2s):
    p = pl.program_id(0)
    b = pl.program_id(1)
    vec = vec_ref[...]
    B, N, _ = adjs.shape
    H = r1ps.shape[2]
    inv_bn = 1.0 / float(B * N)

    @pl.when(p == 0)
    def _phase0():
        adj = adj_ref[...]
        adjs[b] = adj
        x = x_ref[...]
        sum_adj2 = jnp.sum(jnp.sum(adj * adj, axis=1, keepdims=True),
                           axis=0, keepdims=True)
        a2s[b] = sum_adj2 * jnp.ones((8, 128), jnp.float32)
        agg = (jnp.dot(adj, x, preferred_element_type=jnp.float32)
               * _inv_deg(adj))
        cat = jnp.concatenate([agg, x], axis=-1)
        r1p = jnp.maximum(_sage(cat, w1p_ref, vec[0:1]), 0.0)
        r1e = jnp.maximum(_sage(cat, w1e_ref, vec[1:2]), 0.0)
        r1ps[b] = r1p
        r1es[b] = r1e
        part = jnp.concatenate([
            jnp.sum(r1p, axis=0, keepdims=True),
            jnp.sum(r1p * r1p, axis=0, keepdims=True),
            jnp.sum(r1e, axis=0, keepdims=True),
            jnp.sum(r1e * r1e, axis=0, keepdims=True),
            jnp.zeros((4, H), jnp.float32)], axis=0)
        sts[0:8, :] = jnp.where(b == 0, part, sts[0:8, :] + part)

    @pl.when(p == 1)
    def _phase1():
        adj = adjs[b]
        mp, rwp, bbp = _bn_coef(sts[0:2, :], inv_bn, vec[2:3], vec[3:4])
        me, rwe, bbe = _bn_coef(sts[2:4, :], inv_bn, vec[4:5], vec[5:6])
        h1p = (r1ps[b] - mp) * rwp + bbp
        h1e = (r1es[b] - me) * rwe + bbe
        agg = (jnp.dot(adj, jnp.concatenate([h1p, h1e], axis=-1),
                       preferred_element_type=jnp.float32) * _inv_deg(adj))
        r2p = jnp.maximum(
            _sage(jnp.concatenate([agg[:, :H], h1p], axis=-1), w2p_ref,
                  vec[6:7]), 0.0)
        r2e = jnp.maximum(
            _sage(jnp.concatenate([agg[:, H:], h1e], axis=-1), w2e_ref,
                  vec[7:8]), 0.0)
        r2ps[b] = r2p
        r2es[b] = r2e
        part = jnp.concatenate([
            jnp.sum(r2p, axis=0, keepdims=True),
            jnp.sum(r2p * r2p, axis=0, keepdims=True),
            jnp.sum(r2e, axis=0, keepdims=True),
            jnp.sum(r2e * r2e, axis=0, keepdims=True),
            jnp.zeros((4, H), jnp.float32)], axis=0)
        sts[8:16, :] = jnp.where(b == 0, part, sts[8:16, :] + part)

    @pl.when(p == 2)
    def _phase2():
        adj = adjs[b]
        mp, rwp, bbp = _bn_coef(sts[0:2, :], inv_bn, vec[2:3], vec[3:4])
        me, rwe, bbe = _bn_coef(sts[2:4, :], inv_bn, vec[4:5], vec[5:6])
        h1p = (r1ps[b] - mp) * rwp + bbp
        h1e = (r1es[b] - me) * rwe + bbe
        mp, rwp, bbp = _bn_coef(sts[8:10, :], inv_bn, vec[8:9], vec[9:10])
        me, rwe, bbe = _bn_coef(sts[10:12, :], inv_bn, vec[10:11],
                                vec[11:12])
        h2p = (r2ps[b] - mp) * rwp + bbp
        h2e = (r2es[b] - me) * rwe + bbe

        agg = (jnp.dot(adj, jnp.concatenate([h2p, h2e], axis=-1),
                       preferred_element_type=jnp.float32) * _inv_deg(adj))
        h3p = _sage(jnp.concatenate([agg[:, :H], h2p], axis=-1), w3p_ref,
                    vec[12:13])
        h3e = _sage(jnp.concatenate([agg[:, H:], h2e], axis=-1), w3e_ref,
                    vec[13:14])

        logits = (jnp.dot(jnp.concatenate([h1p, h2p, h3p], axis=-1),
                          wlin_ref[...], preferred_element_type=jnp.float32)
                  + vec[14:15])
        m = jnp.max(logits, axis=-1, keepdims=True)
        e = jnp.exp(logits - m)
        sb = e / jnp.sum(e, axis=-1, keepdims=True)
        s_ref[...] = sb

        xb = jnp.concatenate([h1e, h2e, h3e], axis=-1)
        cT = (((0,), (0,)), ((), ()))
        out_x_ref[...] = lax.dot_general(sb, xb, cT,
                                         preferred_element_type=jnp.float32)
        sta = lax.dot_general(sb, adj, cT, preferred_element_type=jnp.float32)
        pooled = jnp.dot(sta, sb, preferred_element_type=jnp.float32)
        sts_mat = lax.dot_general(sb, sb, cT,
                                  preferred_element_type=jnp.float32)

        K = sb.shape[1]
        row = lax.broadcasted_iota(jnp.int32, (K, K), 0)
        col = lax.broadcasted_iota(jnp.int32, (K, K), 1)
        diag = row == col

        sum_adj2 = a2s[b][0:1, 0:1]
        tr_pooled = jnp.sum(jnp.sum(jnp.where(diag, pooled, 0.0),
                                    axis=1, keepdims=True),
                            axis=0, keepdims=True)
        sum_sts2 = jnp.sum(jnp.sum(sts_mat * sts_mat, axis=1, keepdims=True),
                           axis=0, keepdims=True)
        link_ref[...] = sum_adj2 - 2.0 * tr_pooled + sum_sts2

        ent = -sb * jnp.log(sb + _DIFFPOOL_EPS)
        ent_ref[...] = jnp.sum(jnp.sum(ent, axis=1, keepdims=True),
                               axis=0, keepdims=True)

        mn = jnp.min(jnp.min(pooled, axis=1, keepdims=True),
                     axis=0, keepdims=True)
        mx = jnp.max(jnp.max(pooled, axis=1, keepdims=True),
                     axis=0, keepdims=True)
        an = (pooled - mn) / jnp.maximum(mx - mn, 1e-12)
        hard = jnp.where(an + gd_ref[...] >= 1.0 - an, 1.0, 0.0)
        ut = jnp.where(col >= row, hard, 0.0)
        sym = ut + ut.T
        out_adj_ref[...] = jnp.where(diag, 1.0, sym)


def kernel(x, adj, rng, pool_w_rel1, pool_b1, pool_w_root1, pool_w_rel2,
           pool_b2, pool_w_root2, pool_w_rel3, pool_b3, pool_w_root3,
           pool_bn1_w, pool_bn1_b, pool_bn2_w, pool_bn2_b, pool_w_lin,
           pool_b_lin, emb_w_rel1, emb_b1, emb_w_root1, emb_w_rel2, emb_b2,
           emb_w_root2, emb_w_rel3, emb_b3, emb_w_root3, emb_bn1_w,
           emb_bn1_b, emb_bn2_w, emb_bn2_b):
    B, N, C = x.shape
    H = pool_w_rel1.shape[1]
    K = pool_w_lin.shape[1]
    Fe = emb_w_rel3.shape[1]
    D = 2 * H + Fe

    key = jax.random.wrap_key_data(rng)
    g = jax.random.gumbel(key, (2, B, K, K), jnp.float32)
    gd = g[0] - g[1]

    def wcat(wr, wo):
        return jnp.concatenate([wr, wo], axis=0)

    w1p = wcat(pool_w_rel1, pool_w_root1)
    w2p = wcat(pool_w_rel2, pool_w_root2)
    w3p = wcat(pool_w_rel3, pool_w_root3)
    w1e = wcat(emb_w_rel1, emb_w_root1)
    w2e = wcat(emb_w_rel2, emb_w_root2)
    w3e = wcat(emb_w_rel3, emb_w_root3)

    zrow = jnp.zeros((1, H), jnp.float32)
    vec = jnp.concatenate([pool_b1, emb_b1,
                           pool_bn1_w, pool_bn1_b, emb_bn1_w, emb_bn1_b,
                           pool_b2, emb_b2,
                           pool_bn2_w, pool_bn2_b, emb_bn2_w, emb_bn2_b,
                           pool_b3, emb_b3, pool_b_lin, zrow], axis=0)

    def _in0(shape):
        return pl.BlockSpec(shape, lambda p, b: (0,) * len(shape))

    def _phase_blk(phase, park, *shape):
        if phase == 0:
            def imap(p, b):
                return (jnp.where(p == 0, b, park),) + (0,) * len(shape)
        else:
            def imap(p, b):
                return (jnp.where(p == 2, b, 0),) + (0,) * len(shape)
        return pl.BlockSpec((None,) + shape, imap)

    out_x, new_adj, s_soft, link_p, ent_p = pl.pallas_call(
        _mono_body,
        grid=(3, B),
        in_specs=[_phase_blk(0, B - 1, N, C), _phase_blk(0, B - 1, N, N),
                  _phase_blk(2, 0, K, K),
                  _in0((2 * C, H)), _in0((2 * C, H)),
                  _in0((2 * H, H)), _in0((2 * H, H)),
                  _in0((2 * H, H)), _in0((2 * H, H)),
                  _in0((2 * H + K, K)), _in0((16, H))],
        out_specs=(_phase_blk(2, 0, K, D), _phase_blk(2, 0, K, K),
                   _phase_blk(2, 0, N, K), _phase_blk(2, 0, 1, 1),
                   _phase_blk(2, 0, 1, 1)),
        out_shape=(jax.ShapeDtypeStruct((B, K, D), jnp.float32),
                   jax.ShapeDtypeStruct((B, K, K), jnp.float32),
                   jax.ShapeDtypeStruct((B, N, K), jnp.float32),
                   jax.ShapeDtypeStruct((B, 1, 1), jnp.float32),
                   jax.ShapeDtypeStruct((B, 1, 1), jnp.float32)),
        scratch_shapes=[pltpu.VMEM((B, N, N), jnp.float32),
                        pltpu.VMEM((B, N, H), jnp.float32),
                        pltpu.VMEM((B, N, H), jnp.float32),
                        pltpu.VMEM((B, N, H), jnp.float32),
                        pltpu.VMEM((B, N, H), jnp.float32),
                        pltpu.VMEM((16, H), jnp.float32),
                        pltpu.VMEM((B, 8, 128), jnp.float32)],
        compiler_params=pltpu.CompilerParams(
            dimension_semantics=("arbitrary", "arbitrary"),
            vmem_limit_bytes=_VMEM_LIMIT),
    )(x, adj, gd, w1p, w1e, w2p, w2e, w3p, w3e, pool_w_lin, vec)

    link = jnp.sqrt(jnp.maximum(jnp.sum(link_p), 0.0)) / float(B * N * N)
    ent = jnp.sum(ent_p) / float(B * N)
    return out_x, new_adj, link, ent, s_soft


# speedup vs baseline: 1.1864x; 1.0159x over previous
import jax
import jax.numpy as jnp
from jax import lax
from jax.experimental import pallas as pl
from jax.experimental.pallas import tpu as pltpu

_BN_EPS = 1e-5
_NORM_EPS = 1e-12
_DIFFPOOL_EPS = 1e-15
_VMEM_LIMIT = 48 * 1024 * 1024


def _inv_deg(adj):
    return 1.0 / jnp.maximum(jnp.sum(adj, axis=-1, keepdims=True), 1.0)


def _l2norm(out):
    ss = jnp.sum(out * out, axis=-1, keepdims=True)
    return out * lax.rsqrt(jnp.maximum(ss, _NORM_EPS * _NORM_EPS))


def _sage(cat, w_ref, b):
    out = jnp.dot(cat, w_ref[...], preferred_element_type=jnp.float32) + b
    return _l2norm(out)


def _bn_coef(st, inv_bn, w, b):
    mean = st[0:1] * inv_bn
    var = jnp.maximum(st[1:2] * inv_bn - mean * mean, 0.0)
    return mean, lax.rsqrt(var + _BN_EPS) * w, b


_ROT_A = (13, 15, 26, 6)
_ROT_B = (17, 29, 16, 24)


def _rotl(v, r):
    return lax.shift_left(v, r) | lax.shift_right_logical(v, 32 - r)


def _threefry_bits(idx, k1, k2):
    ks2 = k1 ^ k2 ^ 0x1BD11BDA
    sched = (k2, ks2, ks2, k1, k1, k2, k2, ks2, ks2, k1)
    x0 = jnp.zeros_like(idx) + k1
    x1 = idx + k2
    for i in range(5):
        rots = _ROT_A if i % 2 == 0 else _ROT_B
        for r in rots:
            x0 = x0 + x1
            x1 = _rotl(x1, r)
            x1 = x0 ^ x1
        x0 = x0 + sched[2 * i]
        x1 = x1 + sched[2 * i + 1] + (i + 1)
    return x0 ^ x1


def _gumbel_from_bits(bits):
    tiny = jnp.float32(jnp.finfo(jnp.float32).tiny)
    fb = lax.shift_right_logical(bits, 9) | jnp.int32(0x3F800000)
    floats = lax.bitcast_convert_type(fb, jnp.float32) - 1.0
    u = jnp.maximum(tiny, floats * (jnp.float32(1.0) - tiny) + tiny)
    return -jnp.log(-jnp.log(u))


def _mono_body(x_ref, adj_ref, w1pe_ref, w2p_ref, w2e_ref,
               w3p_ref, w3e_ref, wlin_ref, vec_ref, kr_ref,
               out_x_ref, out_adj_ref, s_ref, link_ref, ent_ref,
               adjs, r1ps, r1es, r2ps, r2es, sts, a2s, gds):
    p = pl.program_id(0)
    b = pl.program_id(1)
    vec = vec_ref[...]
    B, N, _ = adjs.shape
    H = r1ps.shape[2]
    K = gds.shape[1]
    inv_bn = 1.0 / float(B * N)

    @pl.when(p == 0)
    def _phase0():
        adj = adj_ref[...]
        adjs[b] = adj
        x = x_ref[...]
        sum_adj2 = jnp.sum(jnp.sum(adj * adj, axis=1, keepdims=True),
                           axis=0, keepdims=True)
        a2s[b] = sum_adj2 * jnp.ones((8, 128), jnp.float32)

        k1 = kr_ref[0:1, 0:1]
        k2 = kr_ref[1:2, 0:1]
        rr = lax.broadcasted_iota(jnp.int32, (K, K), 0)
        cc = lax.broadcasted_iota(jnp.int32, (K, K), 1)
        idx = b * (K * K) + rr * K + cc
        g0 = _gumbel_from_bits(_threefry_bits(idx, k1, k2))
        g1 = _gumbel_from_bits(_threefry_bits(idx + B * K * K, k1, k2))
        gds[b] = g0 - g1

        agg = (jnp.dot(adj, x, preferred_element_type=jnp.float32)
               * _inv_deg(adj))
        cat = jnp.concatenate([agg, x], axis=-1)
        z1 = jnp.dot(cat, w1pe_ref[...], preferred_element_type=jnp.float32)
        r1p = jnp.maximum(_l2norm(z1[:, :H] + vec[0:1]), 0.0)
        r1e = jnp.maximum(_l2norm(z1[:, H:] + vec[1:2]), 0.0)
        r1ps[b] = r1p
        r1es[b] = r1e
        part = jnp.concatenate([
            jnp.sum(r1p, axis=0, keepdims=True),
            jnp.sum(r1p * r1p, axis=0, keepdims=True),
            jnp.sum(r1e, axis=0, keepdims=True),
            jnp.sum(r1e * r1e, axis=0, keepdims=True),
            jnp.zeros((4, H), jnp.float32)], axis=0)
        sts[0:8, :] = jnp.where(b == 0, part, sts[0:8, :] + part)

    @pl.when(p == 1)
    def _phase1():
        adj = adjs[b]
        mp, rwp, bbp = _bn_coef(sts[0:2, :], inv_bn, vec[2:3], vec[3:4])
        me, rwe, bbe = _bn_coef(sts[2:4, :], inv_bn, vec[4:5], vec[5:6])
        h1p = (r1ps[b] - mp) * rwp + bbp
        h1e = (r1es[b] - me) * rwe + bbe
        agg = (jnp.dot(adj, jnp.concatenate([h1p, h1e], axis=-1),
                       preferred_element_type=jnp.float32) * _inv_deg(adj))
        r2p = jnp.maximum(
            _sage(jnp.concatenate([agg[:, :H], h1p], axis=-1), w2p_ref,
                  vec[6:7]), 0.0)
        r2e = jnp.maximum(
            _sage(jnp.concatenate([agg[:, H:], h1e], axis=-1), w2e_ref,
                  vec[7:8]), 0.0)
        r2ps[b] = r2p
        r2es[b] = r2e
        part = jnp.concatenate([
            jnp.sum(r2p, axis=0, keepdims=True),
            jnp.sum(r2p * r2p, axis=0, keepdims=True),
            jnp.sum(r2e, axis=0, keepdims=True),
            jnp.sum(r2e * r2e, axis=0, keepdims=True),
            jnp.zeros((4, H), jnp.float32)], axis=0)
        sts[8:16, :] = jnp.where(b == 0, part, sts[8:16, :] + part)

    @pl.when(p == 2)
    def _phase2():
        adj = adjs[b]
        mp, rwp, bbp = _bn_coef(sts[0:2, :], inv_bn, vec[2:3], vec[3:4])
        me, rwe, bbe = _bn_coef(sts[2:4, :], inv_bn, vec[4:5], vec[5:6])
        h1p = (r1ps[b] - mp) * rwp + bbp
        h1e = (r1es[b] - me) * rwe + bbe
        mp, rwp, bbp = _bn_coef(sts[8:10, :], inv_bn, vec[8:9], vec[9:10])
        me, rwe, bbe = _bn_coef(sts[10:12, :], inv_bn, vec[10:11],
                                vec[11:12])
        h2p = (r2ps[b] - mp) * rwp + bbp
        h2e = (r2es[b] - me) * rwe + bbe

        agg = (jnp.dot(adj, jnp.concatenate([h2p, h2e], axis=-1),
                       preferred_element_type=jnp.float32) * _inv_deg(adj))
        h3p = _sage(jnp.concatenate([agg[:, :H], h2p], axis=-1), w3p_ref,
                    vec[12:13])
        h3e = _sage(jnp.concatenate([agg[:, H:], h2e], axis=-1), w3e_ref,
                    vec[13:14])

        logits = (jnp.dot(jnp.concatenate([h1p, h2p, h3p], axis=-1),
                          wlin_ref[...], preferred_element_type=jnp.float32)
                  + vec[14:15])
        m = jnp.max(logits, axis=-1, keepdims=True)
        e = jnp.exp(logits - m)
        sb = e / jnp.sum(e, axis=-1, keepdims=True)
        s_ref[...] = sb

        xb = jnp.concatenate([h1e, h2e, h3e], axis=-1)
        cT = (((0,), (0,)), ((), ()))
        out_x_ref[...] = lax.dot_general(sb, xb, cT,
                                         preferred_element_type=jnp.float32)
        sta = lax.dot_general(sb, adj, cT, preferred_element_type=jnp.float32)
        pooled = jnp.dot(sta, sb, preferred_element_type=jnp.float32)
        sts_mat = lax.dot_general(sb, sb, cT,
                                  preferred_element_type=jnp.float32)

        K = sb.shape[1]
        row = lax.broadcasted_iota(jnp.int32, (K, K), 0)
        col = lax.broadcasted_iota(jnp.int32, (K, K), 1)
        diag = row == col

        sum_adj2 = a2s[b][0:1, 0:1]
        tr_pooled = jnp.sum(jnp.sum(jnp.where(diag, pooled, 0.0),
                                    axis=1, keepdims=True),
                            axis=0, keepdims=True)
        sum_sts2 = jnp.sum(jnp.sum(sts_mat * sts_mat, axis=1, keepdims=True),
                           axis=0, keepdims=True)
        link_ref[...] = sum_adj2 - 2.0 * tr_pooled + sum_sts2

        ent = -sb * jnp.log(sb + _DIFFPOOL_EPS)
        ent_ref[...] = jnp.sum(jnp.sum(ent, axis=1, keepdims=True),
                               axis=0, keepdims=True)

        mn = jnp.min(jnp.min(pooled, axis=1, keepdims=True),
                     axis=0, keepdims=True)
        mx = jnp.max(jnp.max(pooled, axis=1, keepdims=True),
                     axis=0, keepdims=True)
        an = (pooled - mn) / jnp.maximum(mx - mn, 1e-12)
        hard = jnp.where(an + gds[b] >= 1.0 - an, 1.0, 0.0)
        ut = jnp.where(col >= row, hard, 0.0)
        sym = ut + ut.T
        out_adj_ref[...] = jnp.where(diag, 1.0, sym)


def kernel(x, adj, rng, pool_w_rel1, pool_b1, pool_w_root1, pool_w_rel2,
           pool_b2, pool_w_root2, pool_w_rel3, pool_b3, pool_w_root3,
           pool_bn1_w, pool_bn1_b, pool_bn2_w, pool_bn2_b, pool_w_lin,
           pool_b_lin, emb_w_rel1, emb_b1, emb_w_root1, emb_w_rel2, emb_b2,
           emb_w_root2, emb_w_rel3, emb_b3, emb_w_root3, emb_bn1_w,
           emb_bn1_b, emb_bn2_w, emb_bn2_b):
    B, N, C = x.shape
    H = pool_w_rel1.shape[1]
    K = pool_w_lin.shape[1]
    Fe = emb_w_rel3.shape[1]
    D = 2 * H + Fe

    rngi = rng.view(jnp.int32)
    zi = jnp.zeros((1, 128), jnp.int32)
    kr = jnp.concatenate([zi + rngi[0], zi + rngi[1]] + [zi] * 6, axis=0)

    def wcat(wr, wo):
        return jnp.concatenate([wr, wo], axis=0)

    w1p = wcat(pool_w_rel1, pool_w_root1)
    w2p = wcat(pool_w_rel2, pool_w_root2)
    w3p = wcat(pool_w_rel3, pool_w_root3)
    w1e = wcat(emb_w_rel1, emb_w_root1)
    w2e = wcat(emb_w_rel2, emb_w_root2)
    w3e = wcat(emb_w_rel3, emb_w_root3)
    w1pe = jnp.concatenate([w1p, w1e], axis=1)

    zrow = jnp.zeros((1, H), jnp.float32)
    vec = jnp.concatenate([pool_b1, emb_b1,
                           pool_bn1_w, pool_bn1_b, emb_bn1_w, emb_bn1_b,
                           pool_b2, emb_b2,
                           pool_bn2_w, pool_bn2_b, emb_bn2_w, emb_bn2_b,
                           pool_b3, emb_b3, pool_b_lin, zrow], axis=0)

    def _in0(shape):
        return pl.BlockSpec(shape, lambda p, b: (0,) * len(shape))

    def _phase_blk(phase, park, *shape):
        if phase == 0:
            def imap(p, b):
                return (jnp.where(p == 0, b, park),) + (0,) * len(shape)
        else:
            def imap(p, b):
                return (jnp.where(p == 2, b, 0),) + (0,) * len(shape)
        return pl.BlockSpec((None,) + shape, imap)

    out_x, new_adj, s_soft, link_p, ent_p = pl.pallas_call(
        _mono_body,
        grid=(3, B),
        in_specs=[_phase_blk(0, B - 1, N, C), _phase_blk(0, B - 1, N, N),
                  _in0((2 * C, 2 * H)),
                  _in0((2 * H, H)), _in0((2 * H, H)),
                  _in0((2 * H, H)), _in0((2 * H, H)),
                  _in0((2 * H + K, K)), _in0((16, H)), _in0((8, 128))],
        out_specs=(_phase_blk(2, 0, K, D), _phase_blk(2, 0, K, K),
                   _phase_blk(2, 0, N, K), _phase_blk(2, 0, 1, 1),
                   _phase_blk(2, 0, 1, 1)),
        out_shape=(jax.ShapeDtypeStruct((B, K, D), jnp.float32),
                   jax.ShapeDtypeStruct((B, K, K), jnp.float32),
                   jax.ShapeDtypeStruct((B, N, K), jnp.float32),
                   jax.ShapeDtypeStruct((B, 1, 1), jnp.float32),
                   jax.ShapeDtypeStruct((B, 1, 1), jnp.float32)),
        scratch_shapes=[pltpu.VMEM((B, N, N), jnp.float32),
                        pltpu.VMEM((B, N, H), jnp.float32),
                        pltpu.VMEM((B, N, H), jnp.float32),
                        pltpu.VMEM((B, N, H), jnp.float32),
                        pltpu.VMEM((B, N, H), jnp.float32),
                        pltpu.VMEM((16, H), jnp.float32),
                        pltpu.VMEM((B, 8, 128), jnp.float32),
                        pltpu.VMEM((B, K, K), jnp.float32)],
        compiler_params=pltpu.CompilerParams(
            dimension_semantics=("arbitrary", "arbitrary"),
            vmem_limit_bytes=_VMEM_LIMIT),
    )(x, adj, w1pe, w2p, w2e, w3p, w3e, pool_w_lin, vec, kr)

    link = jnp.sqrt(jnp.maximum(jnp.sum(link_p), 0.0)) / float(B * N * N)
    ent = jnp.sum(ent_p) / float(B * N)
    return out_x, new_adj, link, ent, s_soft


# speedup vs baseline: 1.1899x; 1.0030x over previous
import jax
import jax.numpy as jnp
from jax import lax
from jax.experimental import pallas as pl
from jax.experimental.pallas import tpu as pltpu

_BN_EPS = 1e-5
_NORM_EPS = 1e-12
_DIFFPOOL_EPS = 1e-15
_VMEM_LIMIT = 48 * 1024 * 1024


def _inv_deg(adj):
    return 1.0 / jnp.maximum(jnp.sum(adj, axis=-1, keepdims=True), 1.0)


def _l2norm(out):
    ss = jnp.sum(out * out, axis=-1, keepdims=True)
    return out * lax.rsqrt(jnp.maximum(ss, _NORM_EPS * _NORM_EPS))


def _sage(cat, w_ref, b):
    out = jnp.dot(cat, w_ref[...], preferred_element_type=jnp.float32) + b
    return _l2norm(out)


def _bn_coef(st, inv_bn, w, b):
    mean = st[0:1] * inv_bn
    var = jnp.maximum(st[1:2] * inv_bn - mean * mean, 0.0)
    return mean, lax.rsqrt(var + _BN_EPS) * w, b


_ROT_A = (13, 15, 26, 6)
_ROT_B = (17, 29, 16, 24)


def _rotl(v, r):
    return lax.shift_left(v, r) | lax.shift_right_logical(v, 32 - r)


def _threefry_bits(idx, k1, k2):
    ks2 = k1 ^ k2 ^ 0x1BD11BDA
    sched = (k2, ks2, ks2, k1, k1, k2, k2, ks2, ks2, k1)
    x0 = jnp.zeros_like(idx) + k1
    x1 = idx + k2
    for i in range(5):
        rots = _ROT_A if i % 2 == 0 else _ROT_B
        for r in rots:
            x0 = x0 + x1
            x1 = _rotl(x1, r)
            x1 = x0 ^ x1
        x0 = x0 + sched[2 * i]
        x1 = x1 + sched[2 * i + 1] + (i + 1)
    return x0 ^ x1


def _gumbel_from_bits(bits):
    tiny = jnp.float32(jnp.finfo(jnp.float32).tiny)
    fb = lax.shift_right_logical(bits, 9) | jnp.int32(0x3F800000)
    floats = lax.bitcast_convert_type(fb, jnp.float32) - 1.0
    u = jnp.maximum(tiny, floats * (jnp.float32(1.0) - tiny) + tiny)
    return -jnp.log(-jnp.log(u))


def _mono_body(x_ref, adj_ref, w1pe_ref, w2p_ref, w2e_ref,
               w3p_ref, w3e_ref, wlin_ref, vec_ref, kr_ref,
               out_x_ref, out_adj_ref, s_ref, link_ref, ent_ref,
               adjs, r1ps, r1es, r2ps, r2es, sts, a2s, gds):
    p = pl.program_id(0)
    b = pl.program_id(1)
    vec = vec_ref[...]
    B, N, _ = adjs.shape
    H = r1ps.shape[2]
    K = gds.shape[1]
    inv_bn = 1.0 / float(B * N)

    @pl.when(p == 0)
    def _phase0():
        adj = adj_ref[...]
        adjs[b] = adj
        x = x_ref[...]
        sum_adj2 = jnp.sum(jnp.sum(adj * adj, axis=1, keepdims=True),
                           axis=0, keepdims=True)
        a2s[b] = sum_adj2 * jnp.ones((8, 128), jnp.float32)

        k1 = kr_ref[0:1, 0:1]
        k2 = kr_ref[1:2, 0:1]
        rr = lax.broadcasted_iota(jnp.int32, (K, K), 0)
        cc = lax.broadcasted_iota(jnp.int32, (K, K), 1)
        idx = b * (K * K) + rr * K + cc
        g0 = _gumbel_from_bits(_threefry_bits(idx, k1, k2))
        g1 = _gumbel_from_bits(_threefry_bits(idx + B * K * K, k1, k2))
        gds[b] = g0 - g1

        agg = (jnp.dot(adj, x, preferred_element_type=jnp.float32)
               * _inv_deg(adj))
        cat = jnp.concatenate([agg, x], axis=-1)
        z1 = jnp.dot(cat, w1pe_ref[...], preferred_element_type=jnp.float32)
        r1p = jnp.maximum(_l2norm(z1[:, :H] + vec[0:1]), 0.0)
        r1e = jnp.maximum(_l2norm(z1[:, H:] + vec[1:2]), 0.0)
        r1ps[b] = r1p
        r1es[b] = r1e
        part = jnp.concatenate([
            jnp.sum(r1p, axis=0, keepdims=True),
            jnp.sum(r1p * r1p, axis=0, keepdims=True),
            jnp.sum(r1e, axis=0, keepdims=True),
            jnp.sum(r1e * r1e, axis=0, keepdims=True),
            jnp.zeros((4, H), jnp.float32)], axis=0)
        sts[0:8, :] = jnp.where(b == 0, part, sts[0:8, :] + part)

    @pl.when(p == 1)
    def _phase1():
        adj = adjs[b]
        mp, rwp, bbp = _bn_coef(sts[0:2, :], inv_bn, vec[2:3], vec[3:4])
        me, rwe, bbe = _bn_coef(sts[2:4, :], inv_bn, vec[4:5], vec[5:6])
        h1p = (r1ps[b] - mp) * rwp + bbp
        h1e = (r1es[b] - me) * rwe + bbe
        agg = (jnp.dot(adj, jnp.concatenate([h1p, h1e], axis=-1),
                       preferred_element_type=jnp.float32) * _inv_deg(adj))
        r2p = jnp.maximum(
            _sage(jnp.concatenate([agg[:, :H], h1p], axis=-1), w2p_ref,
                  vec[6:7]), 0.0)
        r2e = jnp.maximum(
            _sage(jnp.concatenate([agg[:, H:], h1e], axis=-1), w2e_ref,
                  vec[7:8]), 0.0)
        r2ps[b] = r2p
        r2es[b] = r2e
        r1ps[b] = h1p
        r1es[b] = h1e
        part = jnp.concatenate([
            jnp.sum(r2p, axis=0, keepdims=True),
            jnp.sum(r2p * r2p, axis=0, keepdims=True),
            jnp.sum(r2e, axis=0, keepdims=True),
            jnp.sum(r2e * r2e, axis=0, keepdims=True),
            jnp.zeros((4, H), jnp.float32)], axis=0)
        sts[8:16, :] = jnp.where(b == 0, part, sts[8:16, :] + part)

    @pl.when(p == 2)
    def _phase2():
        adj = adjs[b]
        h1p = r1ps[b]
        h1e = r1es[b]
        mp, rwp, bbp = _bn_coef(sts[8:10, :], inv_bn, vec[8:9], vec[9:10])
        me, rwe, bbe = _bn_coef(sts[10:12, :], inv_bn, vec[10:11],
                                vec[11:12])
        h2p = (r2ps[b] - mp) * rwp + bbp
        h2e = (r2es[b] - me) * rwe + bbe

        agg = (jnp.dot(adj, jnp.concatenate([h2p, h2e], axis=-1),
                       preferred_element_type=jnp.float32) * _inv_deg(adj))
        h3p = _sage(jnp.concatenate([agg[:, :H], h2p], axis=-1), w3p_ref,
                    vec[12:13])
        h3e = _sage(jnp.concatenate([agg[:, H:], h2e], axis=-1), w3e_ref,
                    vec[13:14])

        logits = (jnp.dot(jnp.concatenate([h1p, h2p, h3p], axis=-1),
                          wlin_ref[...], preferred_element_type=jnp.float32)
                  + vec[14:15])
        m = jnp.max(logits, axis=-1, keepdims=True)
        e = jnp.exp(logits - m)
        sb = e / jnp.sum(e, axis=-1, keepdims=True)
        s_ref[...] = sb

        xb = jnp.concatenate([h1e, h2e, h3e], axis=-1)
        cT = (((0,), (0,)), ((), ()))
        out_x_ref[...] = lax.dot_general(sb, xb, cT,
                                         preferred_element_type=jnp.float32)
        sta = lax.dot_general(sb, adj, cT, preferred_element_type=jnp.float32)
        pooled = jnp.dot(sta, sb, preferred_element_type=jnp.float32)
        sts_mat = lax.dot_general(sb, sb, cT,
                                  preferred_element_type=jnp.float32)

        K = sb.shape[1]
        row = lax.broadcasted_iota(jnp.int32, (K, K), 0)
        col = lax.broadcasted_iota(jnp.int32, (K, K), 1)
        diag = row == col

        sum_adj2 = a2s[b][0:1, 0:1]
        tr_pooled = jnp.sum(jnp.sum(jnp.where(diag, pooled, 0.0),
                                    axis=1, keepdims=True),
                            axis=0, keepdims=True)
        sum_sts2 = jnp.sum(jnp.sum(sts_mat * sts_mat, axis=1, keepdims=True),
                           axis=0, keepdims=True)
        link_ref[...] = sum_adj2 - 2.0 * tr_pooled + sum_sts2

        ent = -sb * jnp.log(sb + _DIFFPOOL_EPS)
        ent_ref[...] = jnp.sum(jnp.sum(ent, axis=1, keepdims=True),
                               axis=0, keepdims=True)

        mn = jnp.min(jnp.min(pooled, axis=1, keepdims=True),
                     axis=0, keepdims=True)
        mx = jnp.max(jnp.max(pooled, axis=1, keepdims=True),
                     axis=0, keepdims=True)
        an = (pooled - mn) / jnp.maximum(mx - mn, 1e-12)
        hard = jnp.where(an + gds[b] >= 1.0 - an, 1.0, 0.0)
        ut = jnp.where(col >= row, hard, 0.0)
        sym = ut + ut.T
        out_adj_ref[...] = jnp.where(diag, 1.0, sym)


def kernel(x, adj, rng, pool_w_rel1, pool_b1, pool_w_root1, pool_w_rel2,
           pool_b2, pool_w_root2, pool_w_rel3, pool_b3, pool_w_root3,
           pool_bn1_w, pool_bn1_b, pool_bn2_w, pool_bn2_b, pool_w_lin,
           pool_b_lin, emb_w_rel1, emb_b1, emb_w_root1, emb_w_rel2, emb_b2,
           emb_w_root2, emb_w_rel3, emb_b3, emb_w_root3, emb_bn1_w,
           emb_bn1_b, emb_bn2_w, emb_bn2_b):
    B, N, C = x.shape
    H = pool_w_rel1.shape[1]
    K = pool_w_lin.shape[1]
    Fe = emb_w_rel3.shape[1]
    D = 2 * H + Fe

    rngi = rng.view(jnp.int32)
    zi = jnp.zeros((1, 128), jnp.int32)
    kr = jnp.concatenate([zi + rngi[0], zi + rngi[1]] + [zi] * 6, axis=0)

    def wcat(wr, wo):
        return jnp.concatenate([wr, wo], axis=0)

    w1p = wcat(pool_w_rel1, pool_w_root1)
    w2p = wcat(pool_w_rel2, pool_w_root2)
    w3p = wcat(pool_w_rel3, pool_w_root3)
    w1e = wcat(emb_w_rel1, emb_w_root1)
    w2e = wcat(emb_w_rel2, emb_w_root2)
    w3e = wcat(emb_w_rel3, emb_w_root3)
    w1pe = jnp.concatenate([w1p, w1e], axis=1)

    zrow = jnp.zeros((1, H), jnp.float32)
    vec = jnp.concatenate([pool_b1, emb_b1,
                           pool_bn1_w, pool_bn1_b, emb_bn1_w, emb_bn1_b,
                           pool_b2, emb_b2,
                           pool_bn2_w, pool_bn2_b, emb_bn2_w, emb_bn2_b,
                           pool_b3, emb_b3, pool_b_lin, zrow], axis=0)

    def _in0(shape):
        return pl.BlockSpec(shape, lambda p, b: (0,) * len(shape))

    def _phase_blk(phase, park, *shape):
        if phase == 0:
            def imap(p, b):
                return (jnp.where(p == 0, b, park),) + (0,) * len(shape)
        else:
            def imap(p, b):
                return (jnp.where(p == 2, b, 0),) + (0,) * len(shape)
        return pl.BlockSpec((None,) + shape, imap)

    out_x, new_adj, s_soft, link_p, ent_p = pl.pallas_call(
        _mono_body,
        grid=(3, B),
        in_specs=[_phase_blk(0, B - 1, N, C), _phase_blk(0, B - 1, N, N),
                  _in0((2 * C, 2 * H)),
                  _in0((2 * H, H)), _in0((2 * H, H)),
                  _in0((2 * H, H)), _in0((2 * H, H)),
                  _in0((2 * H + K, K)), _in0((16, H)), _in0((8, 128))],
        out_specs=(_phase_blk(2, 0, K, D), _phase_blk(2, 0, K, K),
                   _phase_blk(2, 0, N, K), _phase_blk(2, 0, 1, 1),
                   _phase_blk(2, 0, 1, 1)),
        out_shape=(jax.ShapeDtypeStruct((B, K, D), jnp.float32),
                   jax.ShapeDtypeStruct((B, K, K), jnp.float32),
                   jax.ShapeDtypeStruct((B, N, K), jnp.float32),
                   jax.ShapeDtypeStruct((B, 1, 1), jnp.float32),
                   jax.ShapeDtypeStruct((B, 1, 1), jnp.float32)),
        scratch_shapes=[pltpu.VMEM((B, N, N), jnp.float32),
                        pltpu.VMEM((B, N, H), jnp.float32),
                        pltpu.VMEM((B, N, H), jnp.float32),
                        pltpu.VMEM((B, N, H), jnp.float32),
                        pltpu.VMEM((B, N, H), jnp.float32),
                        pltpu.VMEM((16, H), jnp.float32),
                        pltpu.VMEM((B, 8, 128), jnp.float32),
                        pltpu.VMEM((B, K, K), jnp.float32)],
        compiler_params=pltpu.CompilerParams(
            dimension_semantics=("arbitrary", "arbitrary"),
            vmem_limit_bytes=_VMEM_LIMIT),
    )(x, adj, w1pe, w2p, w2e, w3p, w3e, pool_w_lin, vec, kr)

    link = jnp.sqrt(jnp.maximum(jnp.sum(link_p), 0.0)) / float(B * N * N)
    ent = jnp.sum(ent_p) / float(B * N)
    return out_x, new_adj, link, ent, s_soft
